# baseline (speedup 1.0000x reference)
import jax
import jax.numpy as jnp
from jax.experimental import pallas as pl
from jax.experimental.pallas import tpu as pltpu

_B = 32
_C = 384
_N = 196
_NTOT = float(_B * _N)
_EPS = 1e-5
_BF = jnp.bfloat16


def _fused(x_ref, w_ref, v_ref, out_ref, ybf):
    wd = w_ref[0]
    s1 = jnp.zeros((_C, 1), jnp.float32)
    q1 = jnp.zeros((_C, 1), jnp.float32)
    for b in range(_B):
        y1 = jnp.dot(wd, x_ref[b],
                     preferred_element_type=jnp.float32)
        ybf[b] = y1.astype(_BF)
        s1 = s1 + jnp.sum(y1, axis=1, keepdims=True)
        q1 = q1 + jnp.sum(y1 * y1, axis=1, keepdims=True)
    bd = v_ref[:, 0:1]
    mu1n = s1 / _NTOT
    mu1 = mu1n + bd
    var1 = (q1 / _NTOT + 2.0 * bd * mu1n + bd * bd) - mu1 * mu1
    a1 = v_ref[:, 1:2] * jax.lax.rsqrt(var1 + _EPS)
    c1f = (v_ref[:, 2:3] - mu1 * a1) + a1 * bd
    a1b = a1.astype(_BF)
    c1b = c1f.astype(_BF)

    wc = jnp.dot(w_ref[2], w_ref[1],
                 preferred_element_type=jnp.float32).astype(_BF)
    bc = jnp.dot(w_ref[2].astype(jnp.float32), v_ref[:, 3:4],
                 preferred_element_type=jnp.float32) + v_ref[:, 4:5]

    s2 = jnp.zeros((_C, 1), jnp.float32)
    q2 = jnp.zeros((_C, 1), jnp.float32)
    for b in range(_B):
        y2 = ybf[b] * a1b + c1b
        y4 = jnp.dot(wc, y2, preferred_element_type=jnp.float32)
        out_ref[b] = y4
        s2 = s2 + jnp.sum(y4, axis=1, keepdims=True)
        q2 = q2 + jnp.sum(y4 * y4, axis=1, keepdims=True)
    mu2n = s2 / _NTOT
    mu2 = mu2n + bc
    var2 = (q2 / _NTOT + 2.0 * bc * mu2n + bc * bc) - mu2 * mu2
    a2 = v_ref[:, 5:6] * jax.lax.rsqrt(var2 + _EPS)
    c2f = (v_ref[:, 6:7] - mu2 * a2) + a2 * bc

    for b in range(_B):
        out_ref[b] = out_ref[b] * a2 + c2f


def kernel(x, batch, W_down, b_down, g1, be1, W_gcn, b_gcn, W_up, b_up,
           g2, be2, rel_pos):
    del rel_pos
    scale = jnp.asarray(batch, jnp.float32) / 8.0
    xr = x.reshape(_B, _C, _N).astype(_BF)
    wstack = jnp.stack([W_down, W_gcn, W_up]).astype(_BF)
    vpack = jnp.stack([b_down, g1, be1, b_gcn, b_up,
                       g2 * scale, be2 * scale], axis=1).astype(jnp.float32)
    out = pl.pallas_call(
        _fused,
        out_shape=jax.ShapeDtypeStruct((_B, _C, _N), jnp.float32),
        scratch_shapes=[pltpu.VMEM((_B, _C, _N), _BF)],
    )(xr, wstack, vpack)
    return out.reshape(x.shape)


# speedup vs baseline: 1.0089x; 1.0089x over previous
import jax
import jax.numpy as jnp
from jax.experimental import pallas as pl
from jax.experimental.pallas import tpu as pltpu

_B = 32
_C = 384
_N = 196
_NTOT = float(_B * _N)
_EPS = 1e-5
_BF = jnp.bfloat16


def _fused(x_ref, w_ref, v_ref, out_ref, ybf):
    wd = w_ref[0].astype(_BF)
    s1 = jnp.zeros((_C, 1), jnp.float32)
    q1 = jnp.zeros((_C, 1), jnp.float32)
    for b in range(_B):
        y1 = jnp.dot(wd, x_ref[b],
                     preferred_element_type=jnp.float32)
        ybf[b] = y1.astype(_BF)
        s1 = s1 + jnp.sum(y1, axis=1, keepdims=True)
        q1 = q1 + jnp.sum(y1 * y1, axis=1, keepdims=True)
    bd = v_ref[:, 0:1]
    mu1n = s1 / _NTOT
    mu1 = mu1n + bd
    var1 = (q1 / _NTOT + 2.0 * bd * mu1n + bd * bd) - mu1 * mu1
    a1 = v_ref[:, 1:2] * jax.lax.rsqrt(var1 + _EPS)
    c1f = (v_ref[:, 2:3] - mu1 * a1) + a1 * bd
    a1b = a1.astype(_BF)
    c1b = c1f.astype(_BF)

    wu = w_ref[2]
    wc = jnp.dot(wu.astype(_BF), w_ref[1].astype(_BF),
                 preferred_element_type=jnp.float32).astype(_BF)
    bc = jnp.dot(wu, v_ref[:, 3:4], preferred_element_type=jnp.float32) + v_ref[:, 4:5]

    s2 = jnp.zeros((_C, 1), jnp.float32)
    q2 = jnp.zeros((_C, 1), jnp.float32)
    for b in range(_B):
        y2 = ybf[b] * a1b + c1b
        y4 = jnp.dot(wc, y2, preferred_element_type=jnp.float32)
        out_ref[b] = y4
        s2 = s2 + jnp.sum(y4, axis=1, keepdims=True)
        q2 = q2 + jnp.sum(y4 * y4, axis=1, keepdims=True)
    mu2n = s2 / _NTOT
    mu2 = mu2n + bc
    var2 = (q2 / _NTOT + 2.0 * bc * mu2n + bc * bc) - mu2 * mu2
    a2 = v_ref[:, 5:6] * jax.lax.rsqrt(var2 + _EPS)
    c2f = (v_ref[:, 6:7] - mu2 * a2) + a2 * bc

    for b in range(_B):
        out_ref[b] = out_ref[b] * a2 + c2f


def kernel(x, batch, W_down, b_down, g1, be1, W_gcn, b_gcn, W_up, b_up,
           g2, be2, rel_pos):
    del rel_pos
    scale = jnp.asarray(batch, jnp.float32) / 8.0
    xr = x.reshape(_B, _C, _N).astype(_BF)
    wstack = jnp.stack([W_down, W_gcn, W_up])
    vpack = jnp.stack([b_down, g1, be1, b_gcn, b_up,
                       g2 * scale, be2 * scale], axis=1).astype(jnp.float32)
    out = pl.pallas_call(
        _fused,
        out_shape=jax.ShapeDtypeStruct((_B, _C, _N), jnp.float32),
        scratch_shapes=[pltpu.VMEM((_B, _C, _N), _BF)],
    )(xr, wstack, vpack)
    return out.reshape(x.shape)
